# Initial kernel scaffold; baseline (speedup 1.0000x reference)
import functools

import jax
import jax.numpy as jnp
from jax import lax
from jax.experimental import pallas as pl
from jax.experimental.pallas import tpu as pltpu
from jax.experimental.pallas import tpu_sc as plsc

N = 10000
E = 320000
D = 128
H = D // 2
NC = 2
NS = 16
CW = 80
NCH = E // NS // CW
ROWS2D = E // CW
RPT = N // NS
DEG_PAD = 632
NPAD = NS * DEG_PAD

ROW_BLK = 1000


def _sc_mesh():
    return plsc.VectorSubcoreMesh(
        core_axis_name="c", subcore_axis_name="s", num_cores=NC, num_subcores=NS
    )


def _degrees(edges3d, zerosd, ones_cw):
    @functools.partial(
        pl.kernel,
        out_type=jax.ShapeDtypeStruct((NC, NS, DEG_PAD), jnp.float32),
        mesh=_sc_mesh(),
        scratch_types=[
            pltpu.VMEM((NCH, CW), jnp.int32),
            pltpu.VMEM((CW,), jnp.float32),
            pltpu.VMEM_SHARED((NPAD,), jnp.float32),
            pltpu.SemaphoreType.DMA,
        ],
    )
    def deg_kernel(edges_hbm, zerosd_hbm, ones_hbm, out_hbm, idxv, onesv, acc, sem):
        c = lax.axis_index("c")
        s = lax.axis_index("s")
        pltpu.sync_copy(edges_hbm.at[c].at[pl.ds(s * NCH, NCH)], idxv)
        pltpu.sync_copy(ones_hbm, onesv)
        pltpu.sync_copy(zerosd_hbm.at[s], acc.at[pl.ds(s * DEG_PAD, DEG_PAD)])
        plsc.subcore_barrier()

        G = 10

        def group(g, carry):
            base = g * G
            for k in range(G):
                pltpu.async_copy(onesv, acc.at[idxv.at[base + k]], sem, add=True)
            for k in range(G):
                pltpu.make_async_copy(onesv, acc.at[idxv.at[base + k]], sem).wait()
            return carry

        lax.fori_loop(0, NCH // G, group, 0)
        plsc.subcore_barrier()
        pltpu.sync_copy(acc.at[pl.ds(s * DEG_PAD, DEG_PAD)], out_hbm.at[c].at[s])

    return deg_kernel(edges3d, zerosd, ones_cw)


def _aggregate(table2, edges3d, zerosf):
    @functools.partial(
        pl.kernel,
        out_type=jax.ShapeDtypeStruct((NC, N, H), jnp.float32),
        mesh=_sc_mesh(),
        scratch_types=[
            pltpu.VMEM((NCH, CW), jnp.int32),
            pltpu.VMEM((NCH, CW), jnp.int32),
            pltpu.VMEM((CW, H), jnp.float32),
            pltpu.VMEM((CW, H), jnp.float32),
            pltpu.VMEM_SHARED((N, H), jnp.float32),
            pltpu.SemaphoreType.DMA,
            pltpu.SemaphoreType.DMA,
        ],
    )
    def agg_kernel(
        table_hbm, edges_hbm, zeros_hbm, out_hbm,
        srcv, dstv, buf_a, buf_b, acc, sem_a, sem_b,
    ):
        c = lax.axis_index("c")
        s = lax.axis_index("s")
        tbl = table_hbm.at[c]
        pltpu.sync_copy(edges_hbm.at[0].at[pl.ds(s * NCH, NCH)], srcv)
        pltpu.sync_copy(edges_hbm.at[1].at[pl.ds(s * NCH, NCH)], dstv)
        pltpu.sync_copy(zeros_hbm.at[pl.ds(s * RPT, RPT)], acc.at[pl.ds(s * RPT, RPT)])
        plsc.subcore_barrier()

        pltpu.async_copy(tbl.at[srcv.at[0]], buf_a, sem_a)

        def step(jj, carry):
            j = 2 * jj
            pltpu.async_copy(tbl.at[srcv.at[j + 1]], buf_b, sem_b)
            pltpu.make_async_copy(tbl.at[srcv.at[j]], buf_a, sem_a).wait()
            pltpu.sync_copy(buf_a, acc.at[dstv.at[j]], add=True)

            @pl.when(j + 2 < NCH)
            def _():
                pltpu.async_copy(tbl.at[srcv.at[j + 2]], buf_a, sem_a)

            pltpu.make_async_copy(tbl.at[srcv.at[j + 1]], buf_b, sem_b).wait()
            pltpu.sync_copy(buf_b, acc.at[dstv.at[j + 1]], add=True)
            return carry

        lax.fori_loop(0, NCH // 2, step, 0)
        plsc.subcore_barrier()
        pltpu.sync_copy(acc.at[pl.ds(s * RPT, RPT)], out_hbm.at[c].at[pl.ds(s * RPT, RPT)])

    return agg_kernel(table2, edges3d, zerosf)


def _norm(deg_blk):
    return lax.rsqrt(jnp.maximum(deg_blk, 1.0))


def _stage1(features, W1, deg):
    def body(f_ref, w_ref, deg_ref, o_ref):
        x = f_ref[...]
        no = _norm(deg_ref[0])
        o_ref[...] = jnp.dot(x, w_ref[...], preferred_element_type=jnp.float32) * no

    return pl.pallas_call(
        body,
        grid=(NC, N // ROW_BLK),
        in_specs=[
            pl.BlockSpec((ROW_BLK, D), lambda i, j: (j, 0)),
            pl.BlockSpec((D, H), lambda i, j: (0, i)),
            pl.BlockSpec((NC, ROW_BLK, 1), lambda i, j: (0, j, 0)),
        ],
        out_specs=pl.BlockSpec((None, ROW_BLK, H), lambda i, j: (i, j, 0)),
        out_shape=jax.ShapeDtypeStruct((NC, N, H), jnp.float32),
    )(features, W1, deg)


def _stage2(agg, deg, b, W2):
    def body(m_ref, deg_ref, b_ref, w_ref, o_ref):
        m = jnp.concatenate([m_ref[0], m_ref[1]], axis=1)
        ni = _norm(deg_ref[1])
        no = _norm(deg_ref[0])
        h = jnp.maximum(m * ni + b_ref[...], 0.0)
        o_ref[...] = jnp.dot(h, w_ref[...], preferred_element_type=jnp.float32) * no

    return pl.pallas_call(
        body,
        grid=(NC, N // ROW_BLK),
        in_specs=[
            pl.BlockSpec((NC, ROW_BLK, H), lambda i, j: (0, j, 0)),
            pl.BlockSpec((NC, ROW_BLK, 1), lambda i, j: (0, j, 0)),
            pl.BlockSpec((1, D), lambda i, j: (0, 0)),
            pl.BlockSpec((D, H), lambda i, j: (0, i)),
        ],
        out_specs=pl.BlockSpec((None, ROW_BLK, H), lambda i, j: (i, j, 0)),
        out_shape=jax.ShapeDtypeStruct((NC, N, H), jnp.float32),
    )(agg, deg, b, W2)


def _stage3(agg, deg, b):
    def body(m_ref, deg_ref, b_ref, o_ref):
        m = jnp.concatenate([m_ref[0], m_ref[1]], axis=1)
        ni = _norm(deg_ref[1])
        o_ref[...] = jnp.maximum(m * ni + b_ref[...], 0.0)

    return pl.pallas_call(
        body,
        grid=(N // ROW_BLK,),
        in_specs=[
            pl.BlockSpec((NC, ROW_BLK, H), lambda j: (0, j, 0)),
            pl.BlockSpec((NC, ROW_BLK, 1), lambda j: (0, j, 0)),
            pl.BlockSpec((1, D), lambda j: (0, 0)),
        ],
        out_specs=pl.BlockSpec((ROW_BLK, D), lambda j: (j, 0)),
        out_shape=jax.ShapeDtypeStruct((N, D), jnp.float32),
    )(agg, deg, b)


def kernel(features, edge_index, W1, b1, W2, b2):
    edges3d = edge_index.reshape(2, ROWS2D, CW)
    zerosf = jnp.zeros((N, H), jnp.float32)
    zerosd = jnp.zeros((NS, DEG_PAD), jnp.float32)
    ones_cw = jnp.ones((CW,), jnp.float32)

    degp = _degrees(edges3d, zerosd, ones_cw)
    deg = degp.reshape(NC, NPAD)[:, :N].reshape(NC, N, 1)

    h1p = _stage1(features, W1, deg)
    agg1 = _aggregate(h1p, edges3d, zerosf)
    h2p = _stage2(agg1, deg, b1.reshape(1, D), W2)
    agg2 = _aggregate(h2p, edges3d, zerosf)
    return _stage3(agg2, deg, b2.reshape(1, D))


# speedup vs baseline: 9.2787x; 9.2787x over previous
import functools

import jax
import jax.numpy as jnp
from jax import lax
from jax.experimental import pallas as pl
from jax.experimental.pallas import tpu as pltpu
from jax.experimental.pallas import tpu_sc as plsc

N = 10000
E = 320000
D = 128
NC = 2
NS = 16
NW = NC * NS

CW = 100
NCH_A = E // NW // CW
BI = 20
NBLK = NCH_A // BI
NCH_D = E // NS // CW
RPT = N // NS
DEG_PAD = 640
NPAD = NS * DEG_PAD

ROW_BLK = 1000


def _sc_mesh():
    return plsc.VectorSubcoreMesh(
        core_axis_name="c", subcore_axis_name="s", num_cores=NC, num_subcores=NS
    )


def _degrees(edges4d):
    @functools.partial(
        pl.kernel,
        out_type=jax.ShapeDtypeStruct((NC, 1, NPAD), jnp.float32),
        mesh=_sc_mesh(),
        scratch_types=[
            pltpu.VMEM((NCH_D, CW), jnp.int32),
            pltpu.VMEM((112,), jnp.float32),
            pltpu.VMEM((DEG_PAD,), jnp.float32),
            pltpu.VMEM_SHARED((NPAD,), jnp.float32),
            pltpu.SemaphoreType.DMA,
        ],
    )
    def deg_kernel(edges_hbm, out_hbm, idxv, onesv, zv, acc, sem):
        c = lax.axis_index("c")
        s = lax.axis_index("s")
        pltpu.sync_copy(edges_hbm.at[c].at[s], idxv)
        for k in range(7):
            onesv[pl.ds(16 * k, 16)] = jnp.ones((16,), jnp.float32)
            zv[pl.ds(16 * k, 16)] = jnp.zeros((16,), jnp.float32)
        for k in range(7, DEG_PAD // 16):
            zv[pl.ds(16 * k, 16)] = jnp.zeros((16,), jnp.float32)
        pltpu.sync_copy(zv, acc.at[pl.ds(s * DEG_PAD, DEG_PAD)])
        plsc.subcore_barrier()

        ones_cw = onesv.at[pl.ds(0, CW)]
        G = 10

        def group(g, carry):
            base = g * G
            for k in range(G):
                pltpu.async_copy(ones_cw, acc.at[idxv.at[base + k]], sem, add=True)
            for k in range(G):
                pltpu.make_async_copy(ones_cw, acc.at[idxv.at[base + k]], sem).wait()
            return carry

        lax.fori_loop(0, NCH_D // G, group, 0)
        plsc.subcore_barrier()
        pltpu.sync_copy(
            acc.at[pl.ds(s * DEG_PAD, DEG_PAD)],
            out_hbm.at[c].at[0].at[pl.ds(s * DEG_PAD, DEG_PAD)],
        )

    return deg_kernel(edges4d)


def _aggregate(table, edges5w, zerosf):
    @functools.partial(
        pl.kernel,
        out_type=jax.ShapeDtypeStruct((NC, NS, RPT, D), jnp.float32),
        mesh=_sc_mesh(),
        scratch_types=[
            pltpu.VMEM((BI, CW), jnp.int32),
            pltpu.VMEM((BI, CW), jnp.int32),
            pltpu.VMEM((CW, D), jnp.float32),
            pltpu.VMEM((CW, D), jnp.float32),
            pltpu.VMEM_SHARED((N, D), jnp.float32),
            pltpu.SemaphoreType.DMA,
            pltpu.SemaphoreType.DMA,
        ],
    )
    def agg_kernel(
        table_hbm, edges_hbm, zeros_hbm, out_hbm,
        srcv, dstv, buf_a, buf_b, acc, sem_a, sem_b,
    ):
        c = lax.axis_index("c")
        s = lax.axis_index("s")
        p = c * NS + s
        pltpu.sync_copy(zeros_hbm.at[s], acc.at[pl.ds(s * RPT, RPT)])
        plsc.subcore_barrier()

        def blk_body(blk, carry):
            pltpu.sync_copy(edges_hbm.at[0].at[p].at[blk], srcv)
            pltpu.sync_copy(edges_hbm.at[1].at[p].at[blk], dstv)
            pltpu.async_copy(table_hbm.at[srcv.at[0]], buf_a, sem_a)

            def step(jj, carry2):
                i = 2 * jj
                pltpu.async_copy(table_hbm.at[srcv.at[i + 1]], buf_b, sem_b)
                pltpu.make_async_copy(table_hbm.at[srcv.at[i]], buf_a, sem_a).wait()
                pltpu.sync_copy(buf_a, acc.at[dstv.at[i]], add=True)

                @pl.when(i + 2 < BI)
                def _():
                    pltpu.async_copy(table_hbm.at[srcv.at[i + 2]], buf_a, sem_a)

                pltpu.make_async_copy(table_hbm.at[srcv.at[i + 1]], buf_b, sem_b).wait()
                pltpu.sync_copy(buf_b, acc.at[dstv.at[i + 1]], add=True)
                return carry2

            lax.fori_loop(0, BI // 2, step, 0)
            return carry

        lax.fori_loop(0, NBLK, blk_body, 0)
        plsc.subcore_barrier()
        pltpu.sync_copy(acc.at[pl.ds(s * RPT, RPT)], out_hbm.at[c].at[s])

    return agg_kernel(table, edges5w, zerosf)


def _norm(deg_blk):
    return lax.rsqrt(jnp.maximum(deg_blk, 1.0))


def _stage1(features, W1, deg):
    def body(f_ref, w_ref, deg_ref, o_ref):
        no = _norm(deg_ref[0])
        o_ref[...] = (
            jnp.dot(f_ref[...], w_ref[...], preferred_element_type=jnp.float32) * no
        )

    return pl.pallas_call(
        body,
        grid=(N // ROW_BLK,),
        in_specs=[
            pl.BlockSpec((ROW_BLK, D), lambda j: (j, 0)),
            pl.BlockSpec((D, D), lambda j: (0, 0)),
            pl.BlockSpec((NC, ROW_BLK, 1), lambda j: (0, j, 0)),
        ],
        out_specs=pl.BlockSpec((ROW_BLK, D), lambda j: (j, 0)),
        out_shape=jax.ShapeDtypeStruct((N, D), jnp.float32),
    )(features, W1, deg)


def _stage2(agg, deg, b, W2):
    def body(m_ref, deg_ref, b_ref, w_ref, o_ref):
        m = m_ref[0] + m_ref[1]
        ni = _norm(deg_ref[1])
        no = _norm(deg_ref[0])
        h = jnp.maximum(m * ni + b_ref[...], 0.0)
        o_ref[...] = (
            jnp.dot(h, w_ref[...], preferred_element_type=jnp.float32) * no
        )

    return pl.pallas_call(
        body,
        grid=(N // ROW_BLK,),
        in_specs=[
            pl.BlockSpec((NC, ROW_BLK, D), lambda j: (0, j, 0)),
            pl.BlockSpec((NC, ROW_BLK, 1), lambda j: (0, j, 0)),
            pl.BlockSpec((1, D), lambda j: (0, 0)),
            pl.BlockSpec((D, D), lambda j: (0, 0)),
        ],
        out_specs=pl.BlockSpec((ROW_BLK, D), lambda j: (j, 0)),
        out_shape=jax.ShapeDtypeStruct((N, D), jnp.float32),
    )(agg, deg, b, W2)


def _stage3(agg, deg, b):
    def body(m_ref, deg_ref, b_ref, o_ref):
        m = m_ref[0] + m_ref[1]
        ni = _norm(deg_ref[1])
        o_ref[...] = jnp.maximum(m * ni + b_ref[...], 0.0)

    return pl.pallas_call(
        body,
        grid=(N // ROW_BLK,),
        in_specs=[
            pl.BlockSpec((NC, ROW_BLK, D), lambda j: (0, j, 0)),
            pl.BlockSpec((NC, ROW_BLK, 1), lambda j: (0, j, 0)),
            pl.BlockSpec((1, D), lambda j: (0, 0)),
        ],
        out_specs=pl.BlockSpec((ROW_BLK, D), lambda j: (j, 0)),
        out_shape=jax.ShapeDtypeStruct((N, D), jnp.float32),
    )(agg, deg, b)


def kernel(features, edge_index, W1, b1, W2, b2):
    edges4d = edge_index.reshape(2, NS, NCH_D, CW)
    edges5w = edge_index.reshape(2, NW, NBLK, BI, CW)
    zerosf = jnp.zeros((NS, RPT, D), jnp.float32)

    degp = _degrees(edges4d)
    deg = degp.reshape(NC, NPAD)[:, :N].reshape(NC, N, 1)

    h1p = _stage1(features, W1, deg)
    agg1 = _aggregate(h1p, edges5w, zerosf).reshape(NC, N, D)
    h2p = _stage2(agg1, deg, b1.reshape(1, D), W2)
    agg2 = _aggregate(h2p, edges5w, zerosf).reshape(NC, N, D)
    return _stage3(agg2, deg, b2.reshape(1, D))


# speedup vs baseline: 9.6023x; 1.0349x over previous
import functools

import jax
import jax.numpy as jnp
from jax import lax
from jax.experimental import pallas as pl
from jax.experimental.pallas import tpu as pltpu
from jax.experimental.pallas import tpu_sc as plsc

N = 10000
E = 320000
D = 128
NC = 2
NS = 16
NW = NC * NS

CW = 100
CWA = 50
NCH_A = E // NW // CWA
BI = 40
NBLK = NCH_A // BI
NBUF = 4
NCH_D = E // NS // CW
RPT = N // NS
DEG_PAD = 640
NPAD = NS * DEG_PAD

ROW_BLK = 1000


def _sc_mesh():
    return plsc.VectorSubcoreMesh(
        core_axis_name="c", subcore_axis_name="s", num_cores=NC, num_subcores=NS
    )


def _degrees(edges4d):
    @functools.partial(
        pl.kernel,
        out_type=jax.ShapeDtypeStruct((NC, 1, NPAD), jnp.float32),
        mesh=_sc_mesh(),
        scratch_types=[
            pltpu.VMEM((NCH_D, CW), jnp.int32),
            pltpu.VMEM((112,), jnp.float32),
            pltpu.VMEM((DEG_PAD,), jnp.float32),
            pltpu.VMEM_SHARED((NPAD,), jnp.float32),
            pltpu.SemaphoreType.DMA,
        ],
    )
    def deg_kernel(edges_hbm, out_hbm, idxv, onesv, zv, acc, sem):
        c = lax.axis_index("c")
        s = lax.axis_index("s")
        pltpu.sync_copy(edges_hbm.at[c].at[s], idxv)
        for k in range(7):
            onesv[pl.ds(16 * k, 16)] = jnp.ones((16,), jnp.float32)
            zv[pl.ds(16 * k, 16)] = jnp.zeros((16,), jnp.float32)
        for k in range(7, DEG_PAD // 16):
            zv[pl.ds(16 * k, 16)] = jnp.zeros((16,), jnp.float32)
        pltpu.sync_copy(zv, acc.at[pl.ds(s * DEG_PAD, DEG_PAD)])
        plsc.subcore_barrier()

        ones_cw = onesv.at[pl.ds(0, CW)]
        G = 10

        def group(g, carry):
            base = g * G
            for k in range(G):
                pltpu.async_copy(ones_cw, acc.at[idxv.at[base + k]], sem, add=True)
            for k in range(G):
                pltpu.make_async_copy(ones_cw, acc.at[idxv.at[base + k]], sem).wait()
            return carry

        lax.fori_loop(0, NCH_D // G, group, 0)
        plsc.subcore_barrier()
        pltpu.sync_copy(
            acc.at[pl.ds(s * DEG_PAD, DEG_PAD)],
            out_hbm.at[c].at[0].at[pl.ds(s * DEG_PAD, DEG_PAD)],
        )

    return deg_kernel(edges4d)


def _aggregate(table, edges5w, zerosf):
    @functools.partial(
        pl.kernel,
        out_type=jax.ShapeDtypeStruct((NC, NS, RPT, D), jnp.float32),
        mesh=_sc_mesh(),
        scratch_types=[
            pltpu.VMEM((BI, CWA), jnp.int32),
            pltpu.VMEM((BI, CWA), jnp.int32),
            pltpu.VMEM((CWA, D), jnp.float32),
            pltpu.VMEM((CWA, D), jnp.float32),
            pltpu.VMEM((CWA, D), jnp.float32),
            pltpu.VMEM((CWA, D), jnp.float32),
            pltpu.VMEM_SHARED((N, D), jnp.float32),
            pltpu.SemaphoreType.DMA,
            pltpu.SemaphoreType.DMA,
            pltpu.SemaphoreType.DMA,
            pltpu.SemaphoreType.DMA,
            pltpu.SemaphoreType.DMA,
            pltpu.SemaphoreType.DMA,
            pltpu.SemaphoreType.DMA,
            pltpu.SemaphoreType.DMA,
        ],
    )
    def agg_kernel(
        table_hbm, edges_hbm, zeros_hbm, out_hbm,
        srcv, dstv, b0, b1, b2, b3, acc,
        g0, g1, g2, g3, s0, s1, s2, s3,
    ):
        bufs = [b0, b1, b2, b3]
        gsem = [g0, g1, g2, g3]
        ssem = [s0, s1, s2, s3]
        c = lax.axis_index("c")
        s = lax.axis_index("s")
        p = c * NS + s
        pltpu.sync_copy(zeros_hbm.at[s], acc.at[pl.ds(s * RPT, RPT)])
        plsc.subcore_barrier()

        def gfire(i, u):
            pltpu.async_copy(table_hbm.at[srcv.at[i]], bufs[u], gsem[u])

        def gwait(i, u):
            pltpu.make_async_copy(table_hbm.at[srcv.at[i]], bufs[u], gsem[u]).wait()

        def sfire(i, u):
            pltpu.async_copy(bufs[u], acc.at[dstv.at[i]], ssem[u], add=True)

        def swait(i, u):
            pltpu.make_async_copy(bufs[u], acc.at[dstv.at[i]], ssem[u]).wait()

        def blk_body(blk, carry):
            pltpu.sync_copy(edges_hbm.at[0].at[p].at[blk], srcv)
            pltpu.sync_copy(edges_hbm.at[1].at[p].at[blk], dstv)
            for u in range(NBUF - 1):
                gfire(u, u)

            def ring(kk, carry2):
                for u in range(NBUF):
                    i = NBUF * kk + u
                    gwait(i, u)
                    sfire(i, u)
                    if u == 0:
                        @pl.when(kk > 0)
                        def _():
                            swait(NBUF * kk - 1, NBUF - 1)
                    else:
                        swait(i - 1, u - 1)

                    @pl.when(i + NBUF - 1 < BI)
                    def _():
                        gfire(i + NBUF - 1, (u + NBUF - 1) % NBUF)
                return carry2

            lax.fori_loop(0, BI // NBUF, ring, 0)
            swait(BI - 1, NBUF - 1)
            return carry

        lax.fori_loop(0, NBLK, blk_body, 0)
        plsc.subcore_barrier()
        pltpu.sync_copy(acc.at[pl.ds(s * RPT, RPT)], out_hbm.at[c].at[s])

    return agg_kernel(table, edges5w, zerosf)


def _norm(deg_blk):
    return lax.rsqrt(jnp.maximum(deg_blk, 1.0))


def _stage1(features, W1, deg):
    def body(f_ref, w_ref, deg_ref, o_ref):
        no = _norm(deg_ref[0])
        o_ref[...] = (
            jnp.dot(f_ref[...], w_ref[...], preferred_element_type=jnp.float32) * no
        )

    return pl.pallas_call(
        body,
        grid=(N // ROW_BLK,),
        in_specs=[
            pl.BlockSpec((ROW_BLK, D), lambda j: (j, 0)),
            pl.BlockSpec((D, D), lambda j: (0, 0)),
            pl.BlockSpec((NC, ROW_BLK, 1), lambda j: (0, j, 0)),
        ],
        out_specs=pl.BlockSpec((ROW_BLK, D), lambda j: (j, 0)),
        out_shape=jax.ShapeDtypeStruct((N, D), jnp.float32),
    )(features, W1, deg)


def _stage2(agg, deg, b, W2):
    def body(m_ref, deg_ref, b_ref, w_ref, o_ref):
        m = m_ref[0] + m_ref[1]
        ni = _norm(deg_ref[1])
        no = _norm(deg_ref[0])
        h = jnp.maximum(m * ni + b_ref[...], 0.0)
        o_ref[...] = (
            jnp.dot(h, w_ref[...], preferred_element_type=jnp.float32) * no
        )

    return pl.pallas_call(
        body,
        grid=(N // ROW_BLK,),
        in_specs=[
            pl.BlockSpec((NC, ROW_BLK, D), lambda j: (0, j, 0)),
            pl.BlockSpec((NC, ROW_BLK, 1), lambda j: (0, j, 0)),
            pl.BlockSpec((1, D), lambda j: (0, 0)),
            pl.BlockSpec((D, D), lambda j: (0, 0)),
        ],
        out_specs=pl.BlockSpec((ROW_BLK, D), lambda j: (j, 0)),
        out_shape=jax.ShapeDtypeStruct((N, D), jnp.float32),
    )(agg, deg, b, W2)


def _stage3(agg, deg, b):
    def body(m_ref, deg_ref, b_ref, o_ref):
        m = m_ref[0] + m_ref[1]
        ni = _norm(deg_ref[1])
        o_ref[...] = jnp.maximum(m * ni + b_ref[...], 0.0)

    return pl.pallas_call(
        body,
        grid=(N // ROW_BLK,),
        in_specs=[
            pl.BlockSpec((NC, ROW_BLK, D), lambda j: (0, j, 0)),
            pl.BlockSpec((NC, ROW_BLK, 1), lambda j: (0, j, 0)),
            pl.BlockSpec((1, D), lambda j: (0, 0)),
        ],
        out_specs=pl.BlockSpec((ROW_BLK, D), lambda j: (j, 0)),
        out_shape=jax.ShapeDtypeStruct((N, D), jnp.float32),
    )(agg, deg, b)


def kernel(features, edge_index, W1, b1, W2, b2):
    edges4d = edge_index.reshape(2, NS, NCH_D, CW)
    edges5w = edge_index.reshape(2, NW, NBLK, BI, CWA)
    zerosf = jnp.zeros((NS, RPT, D), jnp.float32)

    degp = _degrees(edges4d)
    deg = degp.reshape(NC, NPAD)[:, :N].reshape(NC, N, 1)

    h1p = _stage1(features, W1, deg)
    agg1 = _aggregate(h1p, edges5w, zerosf).reshape(NC, N, D)
    h2p = _stage2(agg1, deg, b1.reshape(1, D), W2)
    agg2 = _aggregate(h2p, edges5w, zerosf).reshape(NC, N, D)
    return _stage3(agg2, deg, b2.reshape(1, D))


# speedup vs baseline: 9.8829x; 1.0292x over previous
import functools

import jax
import jax.numpy as jnp
from jax import lax
from jax.experimental import pallas as pl
from jax.experimental.pallas import tpu as pltpu
from jax.experimental.pallas import tpu_sc as plsc

N = 10000
E = 320000
D = 128
NC = 2
NS = 16
NW = NC * NS

CW = 100
CWA = 100
NCH_A = E // NW // CWA
BI = 20
NBLK = NCH_A // BI
NBUF = 3
RING = 6
TAIL = BI - NBUF * RING
NCH_D = E // NS // CW
RPT = N // NS
DEG_PAD = 640
NPAD = NS * DEG_PAD

ROW_BLK = 1000


def _sc_mesh():
    return plsc.VectorSubcoreMesh(
        core_axis_name="c", subcore_axis_name="s", num_cores=NC, num_subcores=NS
    )


def _degrees(edges4d):
    @functools.partial(
        pl.kernel,
        out_type=jax.ShapeDtypeStruct((NC, 1, NPAD), jnp.float32),
        mesh=_sc_mesh(),
        scratch_types=[
            pltpu.VMEM((NCH_D, CW), jnp.int32),
            pltpu.VMEM((112,), jnp.float32),
            pltpu.VMEM((DEG_PAD,), jnp.float32),
            pltpu.VMEM_SHARED((NPAD,), jnp.float32),
            pltpu.SemaphoreType.DMA,
        ],
    )
    def deg_kernel(edges_hbm, out_hbm, idxv, onesv, zv, acc, sem):
        c = lax.axis_index("c")
        s = lax.axis_index("s")
        pltpu.sync_copy(edges_hbm.at[c].at[s], idxv)
        for k in range(7):
            onesv[pl.ds(16 * k, 16)] = jnp.ones((16,), jnp.float32)
            zv[pl.ds(16 * k, 16)] = jnp.zeros((16,), jnp.float32)
        for k in range(7, DEG_PAD // 16):
            zv[pl.ds(16 * k, 16)] = jnp.zeros((16,), jnp.float32)
        pltpu.sync_copy(zv, acc.at[pl.ds(s * DEG_PAD, DEG_PAD)])
        plsc.subcore_barrier()

        ones_cw = onesv.at[pl.ds(0, CW)]
        G = 10

        def group(g, carry):
            base = g * G
            for k in range(G):
                pltpu.async_copy(ones_cw, acc.at[idxv.at[base + k]], sem, add=True)
            for k in range(G):
                pltpu.make_async_copy(ones_cw, acc.at[idxv.at[base + k]], sem).wait()
            return carry

        lax.fori_loop(0, NCH_D // G, group, 0)
        plsc.subcore_barrier()
        pltpu.sync_copy(
            acc.at[pl.ds(s * DEG_PAD, DEG_PAD)],
            out_hbm.at[c].at[0].at[pl.ds(s * DEG_PAD, DEG_PAD)],
        )

    return deg_kernel(edges4d)


def _aggregate(table, edges5w, zerosf):
    @functools.partial(
        pl.kernel,
        out_type=jax.ShapeDtypeStruct((NC, NS, RPT, D), jnp.float32),
        mesh=_sc_mesh(),
        scratch_types=[
            pltpu.VMEM((BI, CWA), jnp.int32),
            pltpu.VMEM((BI, CWA), jnp.int32),
            pltpu.VMEM((CWA, D), jnp.float32),
            pltpu.VMEM((CWA, D), jnp.float32),
            pltpu.VMEM((CWA, D), jnp.float32),
            pltpu.VMEM_SHARED((N, D), jnp.float32),
            pltpu.SemaphoreType.DMA,
            pltpu.SemaphoreType.DMA,
            pltpu.SemaphoreType.DMA,
            pltpu.SemaphoreType.DMA,
            pltpu.SemaphoreType.DMA,
            pltpu.SemaphoreType.DMA,
        ],
    )
    def agg_kernel(
        table_hbm, edges_hbm, zeros_hbm, out_hbm,
        srcv, dstv, b0, b1, b2, acc,
        g0, g1, g2, s0, s1, s2,
    ):
        bufs = [b0, b1, b2]
        gsem = [g0, g1, g2]
        ssem = [s0, s1, s2]
        c = lax.axis_index("c")
        s = lax.axis_index("s")
        p = c * NS + s
        pltpu.sync_copy(zeros_hbm.at[s], acc.at[pl.ds(s * RPT, RPT)])
        plsc.subcore_barrier()

        def gfire(i, u):
            pltpu.async_copy(table_hbm.at[srcv.at[i]], bufs[u], gsem[u])

        def gwait(i, u):
            pltpu.make_async_copy(table_hbm.at[srcv.at[i]], bufs[u], gsem[u]).wait()

        def sfire(i, u):
            pltpu.async_copy(bufs[u], acc.at[dstv.at[i]], ssem[u], add=True)

        def swait(i, u):
            pltpu.make_async_copy(bufs[u], acc.at[dstv.at[i]], ssem[u]).wait()

        def blk_body(blk, carry):
            pltpu.sync_copy(edges_hbm.at[0].at[p].at[blk], srcv)
            pltpu.sync_copy(edges_hbm.at[1].at[p].at[blk], dstv)
            for u in range(NBUF - 1):
                gfire(u, u)

            def ring(kk, carry2):
                for u in range(NBUF):
                    i = NBUF * kk + u
                    gwait(i, u)
                    sfire(i, u)
                    if u == 0:
                        @pl.when(kk > 0)
                        def _():
                            swait(NBUF * kk - 1, NBUF - 1)
                    else:
                        swait(i - 1, u - 1)

                    @pl.when(i + NBUF - 1 < BI)
                    def _():
                        gfire(i + NBUF - 1, (u + NBUF - 1) % NBUF)
                return carry2

            lax.fori_loop(0, RING, ring, 0)
            for i in range(NBUF * RING, BI):
                u = i % NBUF
                gwait(i, u)
                sfire(i, u)
                swait(i - 1, (i - 1) % NBUF)
            swait(BI - 1, (BI - 1) % NBUF)
            return carry

        lax.fori_loop(0, NBLK, blk_body, 0)
        plsc.subcore_barrier()
        pltpu.sync_copy(acc.at[pl.ds(s * RPT, RPT)], out_hbm.at[c].at[s])

    return agg_kernel(table, edges5w, zerosf)


def _norm(deg_blk):
    return lax.rsqrt(jnp.maximum(deg_blk, 1.0))


def _stage1(features, W1, deg):
    def body(f_ref, w_ref, deg_ref, o_ref):
        no = _norm(deg_ref[0])
        o_ref[...] = (
            jnp.dot(f_ref[...], w_ref[...], preferred_element_type=jnp.float32) * no
        )

    return pl.pallas_call(
        body,
        grid=(N // ROW_BLK,),
        in_specs=[
            pl.BlockSpec((ROW_BLK, D), lambda j: (j, 0)),
            pl.BlockSpec((D, D), lambda j: (0, 0)),
            pl.BlockSpec((NC, ROW_BLK, 1), lambda j: (0, j, 0)),
        ],
        out_specs=pl.BlockSpec((ROW_BLK, D), lambda j: (j, 0)),
        out_shape=jax.ShapeDtypeStruct((N, D), jnp.float32),
    )(features, W1, deg)


def _stage2(agg, deg, b, W2):
    def body(m_ref, deg_ref, b_ref, w_ref, o_ref):
        m = m_ref[0] + m_ref[1]
        ni = _norm(deg_ref[1])
        no = _norm(deg_ref[0])
        h = jnp.maximum(m * ni + b_ref[...], 0.0)
        o_ref[...] = (
            jnp.dot(h, w_ref[...], preferred_element_type=jnp.float32) * no
        )

    return pl.pallas_call(
        body,
        grid=(N // ROW_BLK,),
        in_specs=[
            pl.BlockSpec((NC, ROW_BLK, D), lambda j: (0, j, 0)),
            pl.BlockSpec((NC, ROW_BLK, 1), lambda j: (0, j, 0)),
            pl.BlockSpec((1, D), lambda j: (0, 0)),
            pl.BlockSpec((D, D), lambda j: (0, 0)),
        ],
        out_specs=pl.BlockSpec((ROW_BLK, D), lambda j: (j, 0)),
        out_shape=jax.ShapeDtypeStruct((N, D), jnp.float32),
    )(agg, deg, b, W2)


def _stage3(agg, deg, b):
    def body(m_ref, deg_ref, b_ref, o_ref):
        m = m_ref[0] + m_ref[1]
        ni = _norm(deg_ref[1])
        o_ref[...] = jnp.maximum(m * ni + b_ref[...], 0.0)

    return pl.pallas_call(
        body,
        grid=(N // ROW_BLK,),
        in_specs=[
            pl.BlockSpec((NC, ROW_BLK, D), lambda j: (0, j, 0)),
            pl.BlockSpec((NC, ROW_BLK, 1), lambda j: (0, j, 0)),
            pl.BlockSpec((1, D), lambda j: (0, 0)),
        ],
        out_specs=pl.BlockSpec((ROW_BLK, D), lambda j: (j, 0)),
        out_shape=jax.ShapeDtypeStruct((N, D), jnp.float32),
    )(agg, deg, b)


def kernel(features, edge_index, W1, b1, W2, b2):
    edges4d = edge_index.reshape(2, NS, NCH_D, CW)
    edges5w = edge_index.reshape(2, NW, NBLK, BI, CWA)
    zerosf = jnp.zeros((NS, RPT, D), jnp.float32)

    degp = _degrees(edges4d)
    deg = degp.reshape(NC, NPAD)[:, :N].reshape(NC, N, 1)

    h1p = _stage1(features, W1, deg)
    agg1 = _aggregate(h1p, edges5w, zerosf).reshape(NC, N, D)
    h2p = _stage2(agg1, deg, b1.reshape(1, D), W2)
    agg2 = _aggregate(h2p, edges5w, zerosf).reshape(NC, N, D)
    return _stage3(agg2, deg, b2.reshape(1, D))


# speedup vs baseline: 9.8854x; 1.0003x over previous
import functools

import jax
import jax.numpy as jnp
from jax import lax
from jax.experimental import pallas as pl
from jax.experimental.pallas import tpu as pltpu
from jax.experimental.pallas import tpu_sc as plsc

N = 10000
E = 320000
D = 128
NC = 2
NS = 16
NW = NC * NS

CW = 100
CWA = 100
NCH_A = E // NW // CWA
BI = 20
NBLK = NCH_A // BI
NBUF = 3
RING = 6
TAIL = BI - NBUF * RING
NCH_D = E // NS // CW
RPT = N // NS
DEG_PAD = 640
NPAD = NS * DEG_PAD

ROW_BLK = 1000


def _sc_mesh():
    return plsc.VectorSubcoreMesh(
        core_axis_name="c", subcore_axis_name="s", num_cores=NC, num_subcores=NS
    )


def _degrees(edges4d):
    @functools.partial(
        pl.kernel,
        out_type=jax.ShapeDtypeStruct((NC, 1, NPAD), jnp.float32),
        mesh=_sc_mesh(),
        scratch_types=[
            pltpu.VMEM((NCH_D, CW), jnp.int32),
            pltpu.VMEM((112,), jnp.float32),
            pltpu.VMEM((DEG_PAD,), jnp.float32),
            pltpu.VMEM_SHARED((NPAD,), jnp.float32),
            pltpu.SemaphoreType.DMA,
        ],
    )
    def deg_kernel(edges_hbm, out_hbm, idxv, onesv, zv, acc, sem):
        c = lax.axis_index("c")
        s = lax.axis_index("s")
        pltpu.sync_copy(edges_hbm.at[c].at[s], idxv)
        for k in range(7):
            onesv[pl.ds(16 * k, 16)] = jnp.ones((16,), jnp.float32)
            zv[pl.ds(16 * k, 16)] = jnp.zeros((16,), jnp.float32)
        for k in range(7, DEG_PAD // 16):
            zv[pl.ds(16 * k, 16)] = jnp.zeros((16,), jnp.float32)
        pltpu.sync_copy(zv, acc.at[pl.ds(s * DEG_PAD, DEG_PAD)])
        plsc.subcore_barrier()

        ones_cw = onesv.at[pl.ds(0, CW)]
        G = 25

        def group(g, carry):
            base = g * G
            for k in range(G):
                pltpu.async_copy(ones_cw, acc.at[idxv.at[base + k]], sem, add=True)
            for k in range(G):
                pltpu.make_async_copy(ones_cw, acc.at[idxv.at[base + k]], sem).wait()
            return carry

        lax.fori_loop(0, NCH_D // G, group, 0)
        plsc.subcore_barrier()
        pltpu.sync_copy(
            acc.at[pl.ds(s * DEG_PAD, DEG_PAD)],
            out_hbm.at[c].at[0].at[pl.ds(s * DEG_PAD, DEG_PAD)],
        )

    return deg_kernel(edges4d)


def _aggregate(table, edges5w, zerosf):
    @functools.partial(
        pl.kernel,
        out_type=jax.ShapeDtypeStruct((NC, NS, RPT, D), jnp.float32),
        mesh=_sc_mesh(),
        scratch_types=[
            pltpu.VMEM((BI, CWA), jnp.int32),
            pltpu.VMEM((BI, CWA), jnp.int32),
            pltpu.VMEM((CWA, D), jnp.float32),
            pltpu.VMEM((CWA, D), jnp.float32),
            pltpu.VMEM((CWA, D), jnp.float32),
            pltpu.VMEM_SHARED((N, D), jnp.float32),
            pltpu.SemaphoreType.DMA,
            pltpu.SemaphoreType.DMA,
            pltpu.SemaphoreType.DMA,
            pltpu.SemaphoreType.DMA,
            pltpu.SemaphoreType.DMA,
            pltpu.SemaphoreType.DMA,
        ],
    )
    def agg_kernel(
        table_hbm, edges_hbm, zeros_hbm, out_hbm,
        srcv, dstv, b0, b1, b2, acc,
        g0, g1, g2, s0, s1, s2,
    ):
        bufs = [b0, b1, b2]
        gsem = [g0, g1, g2]
        ssem = [s0, s1, s2]
        c = lax.axis_index("c")
        s = lax.axis_index("s")
        p = c * NS + s
        pltpu.sync_copy(zeros_hbm.at[s], acc.at[pl.ds(s * RPT, RPT)])
        plsc.subcore_barrier()

        def gfire(i, u):
            pltpu.async_copy(table_hbm.at[srcv.at[i]], bufs[u], gsem[u])

        def gwait(i, u):
            pltpu.make_async_copy(table_hbm.at[srcv.at[i]], bufs[u], gsem[u]).wait()

        def sfire(i, u):
            pltpu.async_copy(bufs[u], acc.at[dstv.at[i]], ssem[u], add=True)

        def swait(i, u):
            pltpu.make_async_copy(bufs[u], acc.at[dstv.at[i]], ssem[u]).wait()

        def blk_body(blk, carry):
            pltpu.sync_copy(edges_hbm.at[0].at[p].at[blk], srcv)
            pltpu.sync_copy(edges_hbm.at[1].at[p].at[blk], dstv)
            for u in range(NBUF - 1):
                gfire(u, u)

            def ring(kk, carry2):
                for u in range(NBUF):
                    i = NBUF * kk + u
                    gwait(i, u)
                    sfire(i, u)
                    if u == 0:
                        @pl.when(kk > 0)
                        def _():
                            swait(NBUF * kk - 1, NBUF - 1)
                    else:
                        swait(i - 1, u - 1)

                    @pl.when(i + NBUF - 1 < BI)
                    def _():
                        gfire(i + NBUF - 1, (u + NBUF - 1) % NBUF)
                return carry2

            lax.fori_loop(0, RING, ring, 0)
            for i in range(NBUF * RING, BI):
                u = i % NBUF
                gwait(i, u)
                sfire(i, u)
                swait(i - 1, (i - 1) % NBUF)
            swait(BI - 1, (BI - 1) % NBUF)
            return carry

        lax.fori_loop(0, NBLK, blk_body, 0)
        plsc.subcore_barrier()
        pltpu.sync_copy(acc.at[pl.ds(s * RPT, RPT)], out_hbm.at[c].at[s])

    return agg_kernel(table, edges5w, zerosf)


def _norm(deg_blk):
    return lax.rsqrt(jnp.maximum(deg_blk, 1.0))


def _stage1(features, W1, deg):
    def body(f_ref, w_ref, deg_ref, o_ref):
        no = _norm(deg_ref[0])
        o_ref[...] = (
            jnp.dot(f_ref[...], w_ref[...], preferred_element_type=jnp.float32) * no
        )

    return pl.pallas_call(
        body,
        grid=(N // ROW_BLK,),
        in_specs=[
            pl.BlockSpec((ROW_BLK, D), lambda j: (j, 0)),
            pl.BlockSpec((D, D), lambda j: (0, 0)),
            pl.BlockSpec((NC, ROW_BLK, 1), lambda j: (0, j, 0)),
        ],
        out_specs=pl.BlockSpec((ROW_BLK, D), lambda j: (j, 0)),
        out_shape=jax.ShapeDtypeStruct((N, D), jnp.float32),
    )(features, W1, deg)


def _stage2(agg, deg, b, W2):
    def body(m_ref, deg_ref, b_ref, w_ref, o_ref):
        m = m_ref[0] + m_ref[1]
        ni = _norm(deg_ref[1])
        no = _norm(deg_ref[0])
        h = jnp.maximum(m * ni + b_ref[...], 0.0)
        o_ref[...] = (
            jnp.dot(h, w_ref[...], preferred_element_type=jnp.float32) * no
        )

    return pl.pallas_call(
        body,
        grid=(N // ROW_BLK,),
        in_specs=[
            pl.BlockSpec((NC, ROW_BLK, D), lambda j: (0, j, 0)),
            pl.BlockSpec((NC, ROW_BLK, 1), lambda j: (0, j, 0)),
            pl.BlockSpec((1, D), lambda j: (0, 0)),
            pl.BlockSpec((D, D), lambda j: (0, 0)),
        ],
        out_specs=pl.BlockSpec((ROW_BLK, D), lambda j: (j, 0)),
        out_shape=jax.ShapeDtypeStruct((N, D), jnp.float32),
    )(agg, deg, b, W2)


def _stage3(agg, deg, b):
    def body(m_ref, deg_ref, b_ref, o_ref):
        m = m_ref[0] + m_ref[1]
        ni = _norm(deg_ref[1])
        o_ref[...] = jnp.maximum(m * ni + b_ref[...], 0.0)

    return pl.pallas_call(
        body,
        grid=(N // ROW_BLK,),
        in_specs=[
            pl.BlockSpec((NC, ROW_BLK, D), lambda j: (0, j, 0)),
            pl.BlockSpec((NC, ROW_BLK, 1), lambda j: (0, j, 0)),
            pl.BlockSpec((1, D), lambda j: (0, 0)),
        ],
        out_specs=pl.BlockSpec((ROW_BLK, D), lambda j: (j, 0)),
        out_shape=jax.ShapeDtypeStruct((N, D), jnp.float32),
    )(agg, deg, b)


def kernel(features, edge_index, W1, b1, W2, b2):
    edges4d = edge_index.reshape(2, NS, NCH_D, CW)
    edges5w = edge_index.reshape(2, NW, NBLK, BI, CWA)
    zerosf = jnp.zeros((NS, RPT, D), jnp.float32)

    degp = _degrees(edges4d)
    deg = degp.reshape(NC, NPAD)[:, :N].reshape(NC, N, 1)

    h1p = _stage1(features, W1, deg)
    agg1 = _aggregate(h1p, edges5w, zerosf).reshape(NC, N, D)
    h2p = _stage2(agg1, deg, b1.reshape(1, D), W2)
    agg2 = _aggregate(h2p, edges5w, zerosf).reshape(NC, N, D)
    return _stage3(agg2, deg, b2.reshape(1, D))


# speedup vs baseline: 10.0248x; 1.0141x over previous
import functools

import jax
import jax.numpy as jnp
from jax import lax
from jax.experimental import pallas as pl
from jax.experimental.pallas import tpu as pltpu
from jax.experimental.pallas import tpu_sc as plsc

N = 10000
E = 320000
D = 128
NC = 2
NS = 16
NW = NC * NS

CW = 100
CWA = 100
NCH_A = E // NW // CWA
BI = 20
NBLK = NCH_A // BI
NBUF = 3
RING = 6
TAIL = BI - NBUF * RING
NCH_D = E // NS // CW
RPT = N // NS
DEG_PAD = 640
NPAD = NS * DEG_PAD

ROW_BLK = 1000


def _sc_mesh():
    return plsc.VectorSubcoreMesh(
        core_axis_name="c", subcore_axis_name="s", num_cores=NC, num_subcores=NS
    )


def _degrees(edges4d):
    @functools.partial(
        pl.kernel,
        out_type=jax.ShapeDtypeStruct((NC, 1, NPAD), jnp.float32),
        mesh=_sc_mesh(),
        scratch_types=[
            pltpu.VMEM((NCH_D, CW), jnp.int32),
            pltpu.VMEM((112,), jnp.float32),
            pltpu.VMEM((DEG_PAD,), jnp.float32),
            pltpu.VMEM_SHARED((NPAD,), jnp.float32),
            pltpu.SemaphoreType.DMA,
        ],
    )
    def deg_kernel(edges_hbm, out_hbm, idxv, onesv, zv, acc, sem):
        c = lax.axis_index("c")
        s = lax.axis_index("s")
        pltpu.sync_copy(edges_hbm.at[c].at[s], idxv)
        for k in range(7):
            onesv[pl.ds(16 * k, 16)] = jnp.ones((16,), jnp.float32)
            zv[pl.ds(16 * k, 16)] = jnp.zeros((16,), jnp.float32)
        for k in range(7, DEG_PAD // 16):
            zv[pl.ds(16 * k, 16)] = jnp.zeros((16,), jnp.float32)
        pltpu.sync_copy(zv, acc.at[pl.ds(s * DEG_PAD, DEG_PAD)])
        plsc.subcore_barrier()

        ones_cw = onesv.at[pl.ds(0, CW)]
        G = 25

        def group(g, carry):
            base = g * G
            for k in range(G):
                pltpu.async_copy(ones_cw, acc.at[idxv.at[base + k]], sem, add=True)
            for k in range(G):
                pltpu.make_async_copy(ones_cw, acc.at[idxv.at[base + k]], sem).wait()
            return carry

        lax.fori_loop(0, NCH_D // G, group, 0)
        plsc.subcore_barrier()
        pltpu.sync_copy(
            acc.at[pl.ds(s * DEG_PAD, DEG_PAD)],
            out_hbm.at[c].at[0].at[pl.ds(s * DEG_PAD, DEG_PAD)],
        )

    return deg_kernel(edges4d)


def _aggregate(table, edges5w, zerosf):
    @functools.partial(
        pl.kernel,
        out_type=jax.ShapeDtypeStruct((NC, NS, RPT, D), jnp.float32),
        mesh=_sc_mesh(),
        scratch_types=[
            pltpu.VMEM((BI, CWA), jnp.int32),
            pltpu.VMEM((BI, CWA), jnp.int32),
            pltpu.VMEM((CWA, D), jnp.float32),
            pltpu.VMEM((CWA, D), jnp.float32),
            pltpu.VMEM((CWA, D), jnp.float32),
            pltpu.VMEM_SHARED((N, D), jnp.float32),
            pltpu.SemaphoreType.DMA,
            pltpu.SemaphoreType.DMA,
            pltpu.SemaphoreType.DMA,
            pltpu.SemaphoreType.DMA,
            pltpu.SemaphoreType.DMA,
            pltpu.SemaphoreType.DMA,
            pltpu.SemaphoreType.DMA,
        ],
    )
    def agg_kernel(
        table_hbm, edges_hbm, zeros_hbm, out_hbm,
        srcv, dstv, b0, b1, b2, acc,
        g0, g1, g2, s0, s1, s2, zsem,
    ):
        bufs = [b0, b1, b2]
        gsem = [g0, g1, g2]
        ssem = [s0, s1, s2]
        c = lax.axis_index("c")
        s = lax.axis_index("s")
        p = c * NS + s

        def gfire(i, u):
            pltpu.async_copy(table_hbm.at[srcv.at[i]], bufs[u], gsem[u])

        def gwait(i, u):
            pltpu.make_async_copy(table_hbm.at[srcv.at[i]], bufs[u], gsem[u]).wait()

        def sfire(i, u):
            pltpu.async_copy(bufs[u], acc.at[dstv.at[i]], ssem[u], add=True)

        def swait(i, u):
            pltpu.make_async_copy(bufs[u], acc.at[dstv.at[i]], ssem[u]).wait()

        pltpu.async_copy(zeros_hbm.at[s], acc.at[pl.ds(s * RPT, RPT)], zsem)
        pltpu.sync_copy(edges_hbm.at[0].at[p].at[0], srcv)
        pltpu.sync_copy(edges_hbm.at[1].at[p].at[0], dstv)
        for u in range(NBUF - 1):
            gfire(u, u)
        pltpu.make_async_copy(zeros_hbm.at[s], acc.at[pl.ds(s * RPT, RPT)], zsem).wait()
        plsc.subcore_barrier()

        def blk_body(blk, carry):
            @pl.when(blk > 0)
            def _():
                pltpu.sync_copy(edges_hbm.at[0].at[p].at[blk], srcv)
                pltpu.sync_copy(edges_hbm.at[1].at[p].at[blk], dstv)
                for u in range(NBUF - 1):
                    gfire(u, u)

            def ring(kk, carry2):
                for u in range(NBUF):
                    i = NBUF * kk + u
                    gwait(i, u)
                    sfire(i, u)
                    if u == 0:
                        @pl.when(kk > 0)
                        def _():
                            swait(NBUF * kk - 1, NBUF - 1)
                    else:
                        swait(i - 1, u - 1)

                    @pl.when(i + NBUF - 1 < BI)
                    def _():
                        gfire(i + NBUF - 1, (u + NBUF - 1) % NBUF)
                return carry2

            lax.fori_loop(0, RING, ring, 0)
            for i in range(NBUF * RING, BI):
                u = i % NBUF
                gwait(i, u)
                sfire(i, u)
                swait(i - 1, (i - 1) % NBUF)
            swait(BI - 1, (BI - 1) % NBUF)
            return carry

        lax.fori_loop(0, NBLK, blk_body, 0)
        plsc.subcore_barrier()
        pltpu.sync_copy(acc.at[pl.ds(s * RPT, RPT)], out_hbm.at[c].at[s])

    return agg_kernel(table, edges5w, zerosf)


def _norm(deg_blk):
    return lax.rsqrt(jnp.maximum(deg_blk, 1.0))


def _stage1(features, W1, deg):
    def body(f_ref, w_ref, deg_ref, o_ref):
        no = _norm(deg_ref[0])
        o_ref[...] = (
            jnp.dot(f_ref[...], w_ref[...], preferred_element_type=jnp.float32) * no
        )

    return pl.pallas_call(
        body,
        grid=(N // ROW_BLK,),
        in_specs=[
            pl.BlockSpec((ROW_BLK, D), lambda j: (j, 0)),
            pl.BlockSpec((D, D), lambda j: (0, 0)),
            pl.BlockSpec((NC, ROW_BLK, 1), lambda j: (0, j, 0)),
        ],
        out_specs=pl.BlockSpec((ROW_BLK, D), lambda j: (j, 0)),
        out_shape=jax.ShapeDtypeStruct((N, D), jnp.float32),
    )(features, W1, deg)


def _stage2(agg, deg, b, W2):
    def body(m_ref, deg_ref, b_ref, w_ref, o_ref):
        m = m_ref[0] + m_ref[1]
        ni = _norm(deg_ref[1])
        no = _norm(deg_ref[0])
        h = jnp.maximum(m * ni + b_ref[...], 0.0)
        o_ref[...] = (
            jnp.dot(h, w_ref[...], preferred_element_type=jnp.float32) * no
        )

    return pl.pallas_call(
        body,
        grid=(N // ROW_BLK,),
        in_specs=[
            pl.BlockSpec((NC, ROW_BLK, D), lambda j: (0, j, 0)),
            pl.BlockSpec((NC, ROW_BLK, 1), lambda j: (0, j, 0)),
            pl.BlockSpec((1, D), lambda j: (0, 0)),
            pl.BlockSpec((D, D), lambda j: (0, 0)),
        ],
        out_specs=pl.BlockSpec((ROW_BLK, D), lambda j: (j, 0)),
        out_shape=jax.ShapeDtypeStruct((N, D), jnp.float32),
    )(agg, deg, b, W2)


def _stage3(agg, deg, b):
    def body(m_ref, deg_ref, b_ref, o_ref):
        m = m_ref[0] + m_ref[1]
        ni = _norm(deg_ref[1])
        o_ref[...] = jnp.maximum(m * ni + b_ref[...], 0.0)

    return pl.pallas_call(
        body,
        grid=(N // ROW_BLK,),
        in_specs=[
            pl.BlockSpec((NC, ROW_BLK, D), lambda j: (0, j, 0)),
            pl.BlockSpec((NC, ROW_BLK, 1), lambda j: (0, j, 0)),
            pl.BlockSpec((1, D), lambda j: (0, 0)),
        ],
        out_specs=pl.BlockSpec((ROW_BLK, D), lambda j: (j, 0)),
        out_shape=jax.ShapeDtypeStruct((N, D), jnp.float32),
    )(agg, deg, b)


def kernel(features, edge_index, W1, b1, W2, b2):
    edges4d = edge_index.reshape(2, NS, NCH_D, CW)
    edges5w = edge_index.reshape(2, NW, NBLK, BI, CWA)
    zerosf = jnp.zeros((NS, RPT, D), jnp.float32)

    degp = _degrees(edges4d)
    deg = degp.reshape(NC, NPAD)[:, :N].reshape(NC, N, 1)

    h1p = _stage1(features, W1, deg)
    agg1 = _aggregate(h1p, edges5w, zerosf).reshape(NC, N, D)
    h2p = _stage2(agg1, deg, b1.reshape(1, D), W2)
    agg2 = _aggregate(h2p, edges5w, zerosf).reshape(NC, N, D)
    return _stage3(agg2, deg, b2.reshape(1, D))
